# Initial kernel scaffold; baseline (speedup 1.0000x reference)
import jax
import jax.numpy as jnp
from jax.experimental import pallas as pl


def kernel(batch_graph, adj, Wl1, bl1, Wr1, br1, att1, bias1, Wl2, bl2, Wr2, br2, att2, bias2, Wl3, bl3, Wr3, br3, att3, bias3):
    raise NotImplementedError("write your pallas kernel here")


# speedup vs baseline: 126.4093x; 126.4093x over previous
import jax
import jax.numpy as jnp
from jax.experimental import pallas as pl
from jax.experimental.pallas import tpu as pltpu

N = 256
D = 128
NEG = 0.2
RC = 8


def _gat3_body(x_ref, adjT_ref,
               Wl1, bl1, Wr1, br1, att1, bias1,
               Wl2, bl2, Wr2, br2, att2, bias2,
               Wl3, bl3, Wr3, br3, att3, bias3,
               out_ref, xr_s, S_s):
    mask = adjT_ref[...] > 0.0
    x = x_ref[...]
    for (Wl, bl, Wr, br, att, bias) in (
        (Wl1, bl1, Wr1, br1, att1, bias1),
        (Wl2, bl2, Wr2, br2, att2, bias2),
        (Wl3, bl3, Wr3, br3, att3, bias3),
    ):
        xl = jnp.dot(x, Wl[...], preferred_element_type=jnp.float32) + bl[...]
        xr = jnp.dot(x, Wr[...], preferred_element_type=jnp.float32) + br[...]
        xr_s[...] = xr
        attv = att[...].reshape(1, 1, D)

        def chunk(i, carry):
            xrc = xr_s[pl.ds(i * RC, RC), :]
            m = xrc[:, None, :] + xl[None, :, :]
            t = jnp.maximum(m, NEG * m)
            S_s[pl.ds(i * RC, RC), :] = jnp.sum(t * attv, axis=-1)
            return carry

        jax.lax.fori_loop(0, N // RC, chunk, 0)

        Sm = jnp.where(mask, S_s[...], -jnp.inf)
        amax = jnp.max(Sm, axis=1, keepdims=True)
        amax = jnp.where(amax == -jnp.inf, 0.0, amax)
        e = jnp.exp(Sm - amax)
        denom = jnp.sum(e, axis=1, keepdims=True)
        A = e / (denom + 1e-16)
        x = jnp.dot(A, xl, preferred_element_type=jnp.float32) + bias[...]
    out_ref[...] = x


def kernel(batch_graph, adj, Wl1, bl1, Wr1, br1, att1, bias1,
           Wl2, bl2, Wr2, br2, att2, bias2,
           Wl3, bl3, Wr3, br3, att3, bias3):
    B = batch_graph.shape[0]
    adjT = jnp.swapaxes(adj, 1, 2)
    vecs = [v.reshape(1, -1) for v in (bl1, br1, att1, bias1,
                                       bl2, br2, att2, bias2,
                                       bl3, br3, att3, bias3)]
    (bl1, br1, att1, bias1, bl2, br2, att2, bias2,
     bl3, br3, att3, bias3) = vecs
    weights = (Wl1, bl1, Wr1, br1, att1, bias1,
               Wl2, bl2, Wr2, br2, att2, bias2,
               Wl3, bl3, Wr3, br3, att3, bias3)

    def _full(w):
        return pl.BlockSpec(w.shape, lambda b: (0,) * w.ndim)

    out = pl.pallas_call(
        _gat3_body,
        grid=(B,),
        in_specs=[pl.BlockSpec((None, N, D), lambda b: (b, 0, 0)),
                  pl.BlockSpec((None, N, N), lambda b: (b, 0, 0))]
                 + [_full(w) for w in weights],
        out_specs=pl.BlockSpec((None, N, D), lambda b: (b, 0, 0)),
        out_shape=jax.ShapeDtypeStruct((B, N, D), jnp.float32),
        scratch_shapes=[pltpu.VMEM((N, D), jnp.float32),
                        pltpu.VMEM((N, N), jnp.float32)],
    )(batch_graph, adjT, *weights)
    return out
